# baseline (speedup 1.0000x reference)
import jax
import jax.numpy as jnp
from jax import lax
from jax.experimental import pallas as pl
from jax.experimental.pallas import tpu as pltpu
from jax.experimental.pallas import tpu_sc as plsc

N = 200000
F = 64
C = 256
DELTA_VAR = 0.5
DELTA_DIST = 3.0

NC = 2
NS = 16
NW = NC * NS

ROWS_MAIN = 6240
CHUNK = 416
NCHUNKS = ROWS_MAIN // CHUNK
TAIL_BASE = NW * ROWS_MAIN
NTAIL = (N - TAIL_BASE) // 16

_SC_PARAMS = pltpu.CompilerParams(use_tc_tiling_on_sc=False,
                                  needs_layout_passes=False)
_SC_MESH = dict(core_axis_name="c", subcore_axis_name="s")


def _zero_1d(ref, n):
    zeros = jnp.zeros((16,), jnp.float32)

    def body(i, _):
        ref[pl.ds(i * 16, 16)] = zeros
        return 0

    lax.fori_loop(0, n // 16, body, 0)


def _zero_2d(ref, rows, cols):
    zeros = jnp.zeros((16,), jnp.float32)

    def body(r, _):
        for j in range(cols // 16):
            ref[r, pl.ds(j * 16, 16)] = zeros
        return 0

    lax.fori_loop(0, rows, body, 0)


def _sc_pass1_body(feat_hbm, ids_hbm, psums_hbm, pcnts_hbm,
                   fbuf1, fbuf, ibuf, ftail1, ftail, itail, zrow, cnt2d,
                   acc_sh):
    def rows_1d_to_2d(src1, dst2, nrows):
        def blk_body(b, _):
            r0 = b * 8
            e0 = r0 * F
            for r in range(8):
                for t in range(F // 16):
                    dst2[r0 + r, pl.ds(t * 16, 16)] = (
                        src1[pl.ds(e0 + r * F + t * 16, 16)])
            return 0

        lax.fori_loop(0, nrows // 8, blk_body, 0)

    cid = lax.axis_index("c")
    sid = lax.axis_index("s")
    wid = sid * NC + cid
    base = wid * ROWS_MAIN

    lane = lax.iota(jnp.int32, 16)
    ones = jnp.full((16,), 1.0, jnp.float32)

    _zero_2d(cnt2d, 16, C)
    _zero_2d(zrow, 16, F)
    pltpu.sync_copy(zrow, acc_sh.at[pl.ds(sid * 16, 16)])
    plsc.subcore_barrier()

    def chunk_body(k, _):
        rowbase = base + k * CHUNK
        pltpu.sync_copy(feat_hbm.at[pl.ds(rowbase * F, CHUNK * F)], fbuf1)
        pltpu.sync_copy(ids_hbm.at[pl.ds(rowbase, CHUNK)], ibuf)
        rows_1d_to_2d(fbuf1, fbuf, CHUNK)

        def grp_body(g, _):
            iv = ibuf[pl.ds(g * 16, 16)]
            plsc.addupdate_scatter(cnt2d, [lane, iv], ones)
            return 0

        lax.fori_loop(0, CHUNK // 16, grp_body, 0)
        pltpu.sync_copy(fbuf, acc_sh.at[ibuf], add=True)
        return 0

    lax.fori_loop(0, NCHUNKS, chunk_body, 0)

    @pl.when(wid < NTAIL)
    def _():
        tb = TAIL_BASE + wid * 16
        pltpu.sync_copy(feat_hbm.at[pl.ds(tb * F, 16 * F)], ftail1)
        pltpu.sync_copy(ids_hbm.at[pl.ds(tb, 16)], itail)
        rows_1d_to_2d(ftail1, ftail, 16)
        iv = itail[...]
        plsc.addupdate_scatter(cnt2d, [lane, iv], ones)
        pltpu.sync_copy(ftail, acc_sh.at[itail], add=True)

    plsc.subcore_barrier()

    @pl.when(sid == 0)
    def _():
        pltpu.sync_copy(acc_sh, psums_hbm.at[cid])

    pltpu.sync_copy(cnt2d, pcnts_hbm.at[wid])


def _sc_pass1(feats1d, ids):
    kfn = pl.kernel(
        _sc_pass1_body,
        out_type=(jax.ShapeDtypeStruct((NC, C, F), jnp.float32),
                  jax.ShapeDtypeStruct((NW, 16, C), jnp.float32)),
        mesh=plsc.VectorSubcoreMesh(**_SC_MESH),
        compiler_params=_SC_PARAMS,
        scratch_types=[
            pltpu.VMEM((CHUNK * F,), jnp.float32),
            pltpu.VMEM((CHUNK, F), jnp.float32),
            pltpu.VMEM((CHUNK,), jnp.int32),
            pltpu.VMEM((16 * F,), jnp.float32),
            pltpu.VMEM((16, F), jnp.float32),
            pltpu.VMEM((16,), jnp.int32),
            pltpu.VMEM((16, F), jnp.float32),
            pltpu.VMEM((16, C), jnp.float32),
            pltpu.VMEM_SHARED((C, F), jnp.float32),
        ],
    )
    return kfn(feats1d, ids)


def _sc_pass2_body(feat_hbm, ids_hbm, means_hbm, ppen_hbm,
                   fbuf, ibuf, ftail, itail, means_v, pen2d):
    cid = lax.axis_index("c")
    sid = lax.axis_index("s")
    wid = sid * NC + cid
    base = wid * ROWS_MAIN

    lane = lax.iota(jnp.int32, 16)

    pltpu.sync_copy(means_hbm, means_v)
    _zero_2d(pen2d, 16, C)

    def process_group(fref, iv, r0):
        rows64 = (lane + r0) * F
        iv64 = iv * F
        accs = [jnp.zeros((16,), jnp.float32) for _ in range(4)]
        for jj in range(F // 4):
            for t in range(4):
                jrot = jnp.bitwise_and(lane + (t * (F // 4) + jj), F - 1)
                fcol = plsc.load_gather(fref, [rows64 + jrot])
                mcol = plsc.load_gather(means_v, [iv64 + jrot])
                d = fcol - mcol
                accs[t] = accs[t] + d * d
        nsq = (accs[0] + accs[1]) + (accs[2] + accs[3]) + 1e-12
        i = plsc.bitcast(nsq, jnp.int32)
        i = jnp.int32(0x5F3759DF) - lax.shift_right_arithmetic(i, jnp.int32(1))
        y = plsc.bitcast(i, jnp.float32)
        for _ in range(3):
            y = y * (1.5 - 0.5 * nsq * y * y)
        dist = nsq * y
        pen = jnp.maximum(dist - DELTA_VAR, 0.0)
        plsc.addupdate_scatter(pen2d, [lane, iv], pen * pen)

    def chunk_body(k, _):
        rowbase = base + k * CHUNK
        pltpu.sync_copy(feat_hbm.at[pl.ds(rowbase * F, CHUNK * F)], fbuf)
        pltpu.sync_copy(ids_hbm.at[pl.ds(rowbase, CHUNK)], ibuf)

        def grp_body(g):
            r0 = g * 16
            process_group(fbuf, ibuf[pl.ds(r0, 16)], r0)

        plsc.parallel_loop(0, CHUNK // 16, 1, unroll=2, carry=None)(grp_body)
        return 0

    lax.fori_loop(0, NCHUNKS, chunk_body, 0)

    @pl.when(wid < NTAIL)
    def _():
        tb = TAIL_BASE + wid * 16
        pltpu.sync_copy(feat_hbm.at[pl.ds(tb * F, 16 * F)], ftail)
        pltpu.sync_copy(ids_hbm.at[pl.ds(tb, 16)], itail)
        process_group(ftail, itail[...], 0)

    pltpu.sync_copy(pen2d, ppen_hbm.at[wid])


def _sc_pass2(feats1d, ids, means1d):
    kfn = pl.kernel(
        _sc_pass2_body,
        out_type=jax.ShapeDtypeStruct((NW, 16, C), jnp.float32),
        mesh=plsc.VectorSubcoreMesh(**_SC_MESH),
        compiler_params=_SC_PARAMS,
        scratch_types=[
            pltpu.VMEM((CHUNK * F,), jnp.float32),
            pltpu.VMEM((CHUNK,), jnp.int32),
            pltpu.VMEM((16 * F,), jnp.float32),
            pltpu.VMEM((16,), jnp.int32),
            pltpu.VMEM((C * F,), jnp.float32),
            pltpu.VMEM((16, C), jnp.float32),
        ],
    )
    return kfn(feats1d, ids, means1d)


def _tc_reduce_body(psums_ref, pcnts_ref, means_ref, cnts_ref, dl_ref):
    sums = psums_ref[0] + psums_ref[1]
    cnts = jnp.sum(pcnts_ref[...], axis=(0, 1))
    means = sums / cnts[:, None]
    means_ref[...] = means
    cnts_ref[...] = cnts.reshape(1, C)

    q = jnp.sum(means * means, axis=1)
    g = lax.dot_general(means, means, (((1,), (1,)), ((), ())),
                        preferred_element_type=jnp.float32)
    md2 = jnp.maximum(q[:, None] + q[None, :] - 2.0 * g, 0.0)
    r = lax.broadcasted_iota(jnp.int32, (C, C), 0)
    c = lax.broadcasted_iota(jnp.int32, (C, C), 1)
    eye = (r == c).astype(jnp.float32)
    d = jnp.sqrt(md2 + eye)
    pen = jnp.square(jnp.maximum(DELTA_DIST - d, 0.0)) * (1.0 - eye)
    dl_ref[...] = (jnp.sum(pen) / (C * (C - 1))).reshape(1, 1)


def _tc_reduce(psums, pcnts):
    return pl.pallas_call(
        _tc_reduce_body,
        out_shape=(jax.ShapeDtypeStruct((C, F), jnp.float32),
                   jax.ShapeDtypeStruct((1, C), jnp.float32),
                   jax.ShapeDtypeStruct((1, 1), jnp.float32)),
    )(psums, pcnts)


def _tc_final_body(ppen_ref, cnts_ref, dl_ref, out_ref):
    pen = jnp.sum(ppen_ref[...], axis=(0, 1))
    var_loss = jnp.sum(pen / cnts_ref[0, :]) / C
    out_ref[...] = (var_loss + dl_ref[0, 0]).reshape(1, 1)


def _tc_final(ppen, cnts, dl):
    return pl.pallas_call(
        _tc_final_body,
        out_shape=jax.ShapeDtypeStruct((1, 1), jnp.float32),
    )(ppen, cnts, dl)


def kernel(features, labels):
    feats1d = jnp.reshape(features, (-1,))
    ids = labels[:, 1]
    psums, pcnts = _sc_pass1(feats1d, ids)
    means, cnts, dl = _tc_reduce(psums, pcnts)
    ppen = _sc_pass2(feats1d, ids, jnp.reshape(means, (-1,)))
    out = _tc_final(ppen, cnts, dl)
    return out[0, 0]


# speedup vs baseline: 1.3339x; 1.3339x over previous
import jax
import jax.numpy as jnp
from jax import lax
from jax.experimental import layout as jex_layout
from jax.experimental import pallas as pl
from jax.experimental.pallas import tpu as pltpu
from jax.experimental.pallas import tpu_sc as plsc

N = 200000
F = 64
C = 256
DELTA_VAR = 0.5
DELTA_DIST = 3.0

NC = 2
NS = 16
NW = NC * NS

ROWS_MAIN = 6240
CHUNK = 416
NCHUNKS = ROWS_MAIN // CHUNK
TAIL_BASE = NW * ROWS_MAIN
NTAIL = (N - TAIL_BASE) // 16

_SC_PARAMS = pltpu.CompilerParams(use_tc_tiling_on_sc=False,
                                  needs_layout_passes=False)
_SC_MESH = dict(core_axis_name="c", subcore_axis_name="s")


def _zero_1d(ref, n):
    zeros = jnp.zeros((16,), jnp.float32)

    def body(i, _):
        ref[pl.ds(i * 16, 16)] = zeros
        return 0

    lax.fori_loop(0, n // 16, body, 0)


def _zero_2d(ref, rows, cols):
    zeros = jnp.zeros((16,), jnp.float32)

    def body(r, _):
        for j in range(cols // 16):
            ref[r, pl.ds(j * 16, 16)] = zeros
        return 0

    lax.fori_loop(0, rows, body, 0)


def _sc_pass1_body(feat_hbm, ids_hbm, psums_hbm, pcnts_hbm,
                   fbuf1, fbuf, ibuf, ftail1, ftail, itail, zrow, cnt2d,
                   acc_sh):
    def rows_1d_to_2d(src1, dst2, nrows):
        def blk_body(b, _):
            r0 = b * 8
            e0 = r0 * F
            for r in range(8):
                for t in range(F // 16):
                    dst2[r0 + r, pl.ds(t * 16, 16)] = (
                        src1[pl.ds(e0 + r * F + t * 16, 16)])
            return 0

        lax.fori_loop(0, nrows // 8, blk_body, 0)

    cid = lax.axis_index("c")
    sid = lax.axis_index("s")
    wid = sid * NC + cid
    base = wid * ROWS_MAIN

    lane = lax.iota(jnp.int32, 16)
    ones = jnp.full((16,), 1.0, jnp.float32)

    _zero_2d(cnt2d, 16, C)
    _zero_2d(zrow, 16, F)
    pltpu.sync_copy(zrow, acc_sh.at[pl.ds(sid * 16, 16)])
    plsc.subcore_barrier()

    def chunk_body(k, _):
        rowbase = base + k * CHUNK
        pltpu.sync_copy(feat_hbm.at[pl.ds(rowbase * F, CHUNK * F)], fbuf1)
        pltpu.sync_copy(ids_hbm.at[pl.ds(rowbase, CHUNK)], ibuf)
        rows_1d_to_2d(fbuf1, fbuf, CHUNK)

        def grp_body(g, _):
            iv = ibuf[pl.ds(g * 16, 16)]
            plsc.addupdate_scatter(cnt2d, [lane, iv], ones)
            return 0

        lax.fori_loop(0, CHUNK // 16, grp_body, 0)
        pltpu.sync_copy(fbuf, acc_sh.at[ibuf], add=True)
        return 0

    lax.fori_loop(0, NCHUNKS, chunk_body, 0)

    @pl.when(wid < NTAIL)
    def _():
        tb = TAIL_BASE + wid * 16
        pltpu.sync_copy(feat_hbm.at[pl.ds(tb * F, 16 * F)], ftail1)
        pltpu.sync_copy(ids_hbm.at[pl.ds(tb, 16)], itail)
        rows_1d_to_2d(ftail1, ftail, 16)
        iv = itail[...]
        plsc.addupdate_scatter(cnt2d, [lane, iv], ones)
        pltpu.sync_copy(ftail, acc_sh.at[itail], add=True)

    plsc.subcore_barrier()

    @pl.when(sid == 0)
    def _():
        pltpu.sync_copy(acc_sh, psums_hbm.at[cid])

    pltpu.sync_copy(cnt2d, pcnts_hbm.at[wid])


def _sc_pass1(feats1d, ids):
    kfn = pl.kernel(
        _sc_pass1_body,
        out_type=(jax.ShapeDtypeStruct((NC, C, F), jnp.float32),
                  jax.ShapeDtypeStruct((NW, 16, C), jnp.float32)),
        mesh=plsc.VectorSubcoreMesh(**_SC_MESH),
        compiler_params=_SC_PARAMS,
        scratch_types=[
            pltpu.VMEM((CHUNK * F,), jnp.float32),
            pltpu.VMEM((CHUNK, F), jnp.float32),
            pltpu.VMEM((CHUNK,), jnp.int32),
            pltpu.VMEM((16 * F,), jnp.float32),
            pltpu.VMEM((16, F), jnp.float32),
            pltpu.VMEM((16,), jnp.int32),
            pltpu.VMEM((16, F), jnp.float32),
            pltpu.VMEM((16, C), jnp.float32),
            pltpu.VMEM_SHARED((C, F), jnp.float32),
        ],
    )
    return kfn(feats1d, ids)


def _sc_pass2_body(feat_hbm, ids_hbm, means_hbm, ppen_hbm,
                   fbuf, ibuf, ftail, itail, means_v, pen2d):
    cid = lax.axis_index("c")
    sid = lax.axis_index("s")
    wid = sid * NC + cid
    base = wid * ROWS_MAIN

    lane = lax.iota(jnp.int32, 16)

    pltpu.sync_copy(means_hbm, means_v)
    _zero_2d(pen2d, 16, C)

    def process_group(fref, iv, r0):
        rows64 = (lane + r0) * F
        iv64 = iv * F
        accs = [jnp.zeros((16,), jnp.float32) for _ in range(4)]
        for jj in range(F // 4):
            for t in range(4):
                jrot = jnp.bitwise_and(lane + (t * (F // 4) + jj), F - 1)
                fcol = plsc.load_gather(fref, [rows64 + jrot])
                mcol = plsc.load_gather(means_v, [iv64 + jrot])
                d = fcol - mcol
                accs[t] = accs[t] + d * d
        nsq = (accs[0] + accs[1]) + (accs[2] + accs[3]) + 1e-12
        i = plsc.bitcast(nsq, jnp.int32)
        i = jnp.int32(0x5F3759DF) - lax.shift_right_arithmetic(i, jnp.int32(1))
        y = plsc.bitcast(i, jnp.float32)
        for _ in range(3):
            y = y * (1.5 - 0.5 * nsq * y * y)
        dist = nsq * y
        pen = jnp.maximum(dist - DELTA_VAR, 0.0)
        plsc.addupdate_scatter(pen2d, [lane, iv], pen * pen)

    def chunk_body(k, _):
        rowbase = base + k * CHUNK
        pltpu.sync_copy(feat_hbm.at[pl.ds(rowbase * F, CHUNK * F)], fbuf)
        pltpu.sync_copy(ids_hbm.at[pl.ds(rowbase, CHUNK)], ibuf)

        def grp_body(g, _):
            r0 = g * 16
            process_group(fbuf, ibuf[pl.ds(r0, 16)], r0)
            return 0

        lax.fori_loop(0, CHUNK // 16, grp_body, 0)
        return 0

    lax.fori_loop(0, NCHUNKS, chunk_body, 0)

    @pl.when(wid < NTAIL)
    def _():
        tb = TAIL_BASE + wid * 16
        pltpu.sync_copy(feat_hbm.at[pl.ds(tb * F, 16 * F)], ftail)
        pltpu.sync_copy(ids_hbm.at[pl.ds(tb, 16)], itail)
        process_group(ftail, itail[...], 0)

    pltpu.sync_copy(pen2d, ppen_hbm.at[wid])


def _sc_pass2(feats1d, ids, means1d):
    kfn = pl.kernel(
        _sc_pass2_body,
        out_type=jax.ShapeDtypeStruct((NW, 16, C), jnp.float32),
        mesh=plsc.VectorSubcoreMesh(**_SC_MESH),
        compiler_params=_SC_PARAMS,
        scratch_types=[
            pltpu.VMEM((CHUNK * F,), jnp.float32),
            pltpu.VMEM((CHUNK,), jnp.int32),
            pltpu.VMEM((16 * F,), jnp.float32),
            pltpu.VMEM((16,), jnp.int32),
            pltpu.VMEM((C * F,), jnp.float32),
            pltpu.VMEM((16, C), jnp.float32),
        ],
    )
    return kfn(feats1d, ids, means1d)


def _tc_reduce_body(psums_ref, pcnts_ref, means_ref, cnts_ref, dl_ref):
    sums = psums_ref[0] + psums_ref[1]
    cnts = jnp.sum(pcnts_ref[...], axis=(0, 1))
    means = sums / cnts[:, None]
    means_ref[...] = means
    cnts_ref[...] = cnts.reshape(1, C)

    q = jnp.sum(means * means, axis=1)
    g = lax.dot_general(means, means, (((1,), (1,)), ((), ())),
                        preferred_element_type=jnp.float32)
    md2 = jnp.maximum(q[:, None] + q[None, :] - 2.0 * g, 0.0)
    r = lax.broadcasted_iota(jnp.int32, (C, C), 0)
    c = lax.broadcasted_iota(jnp.int32, (C, C), 1)
    eye = (r == c).astype(jnp.float32)
    d = jnp.sqrt(md2 + eye)
    pen = jnp.square(jnp.maximum(DELTA_DIST - d, 0.0)) * (1.0 - eye)
    dl_ref[...] = (jnp.sum(pen) / (C * (C - 1))).reshape(1, 1)


def _tc_reduce(psums, pcnts):
    return pl.pallas_call(
        _tc_reduce_body,
        out_shape=(jax.ShapeDtypeStruct((C, F), jnp.float32),
                   jax.ShapeDtypeStruct((1, C), jnp.float32),
                   jax.ShapeDtypeStruct((1, 1), jnp.float32)),
    )(psums, pcnts)


def _tc_final_body(ppen_ref, cnts_ref, dl_ref, out_ref):
    pen = jnp.sum(ppen_ref[...], axis=(0, 1))
    var_loss = jnp.sum(pen / cnts_ref[0, :]) / C
    out_ref[...] = (var_loss + dl_ref[0, 0]).reshape(1, 1)


def _tc_final(ppen, cnts, dl):
    return pl.pallas_call(
        _tc_final_body,
        out_shape=jax.ShapeDtypeStruct((1, 1), jnp.float32),
    )(ppen, cnts, dl)


def kernel(features, labels):
    features = jex_layout.with_layout_constraint(
        features, jex_layout.Layout((1, 0)))
    feats1d = jnp.reshape(features, (-1,))
    ids = labels[:, 1]
    psums, pcnts = _sc_pass1(feats1d, ids)
    means, cnts, dl = _tc_reduce(psums, pcnts)
    ppen = _sc_pass2(feats1d, ids, jnp.reshape(means, (-1,)))
    out = _tc_final(ppen, cnts, dl)
    return out[0, 0]


# speedup vs baseline: 1.6288x; 1.2210x over previous
import jax
import jax.numpy as jnp
from jax import lax
from jax.experimental import pallas as pl
from jax.experimental.pallas import tpu as pltpu
from jax.experimental.pallas import tpu_sc as plsc

N = 200000
F = 64
C = 256
DELTA_VAR = 0.5
DELTA_DIST = 3.0

NC = 2
NS = 16
NW = NC * NS

ROWS_MAIN = 6240
CHUNK = 208
NCHUNKS = ROWS_MAIN // CHUNK
TAIL_BASE = NW * ROWS_MAIN
NTAIL = (N - TAIL_BASE) // 16

_SC_PARAMS = pltpu.CompilerParams(use_tc_tiling_on_sc=False,
                                  needs_layout_passes=False)
_SC_MESH = dict(core_axis_name="c", subcore_axis_name="s")


def _zero_1d(ref, n):
    zeros = jnp.zeros((16,), jnp.float32)

    def body(i, _):
        ref[pl.ds(i * 16, 16)] = zeros
        return 0

    lax.fori_loop(0, n // 16, body, 0)


def _zero_2d(ref, rows, cols):
    zeros = jnp.zeros((16,), jnp.float32)

    def body(r, _):
        for j in range(cols // 16):
            ref[r, pl.ds(j * 16, 16)] = zeros
        return 0

    lax.fori_loop(0, rows, body, 0)


def _sc_pass1_body(feat_hbm, ids_hbm, psums_hbm, pcnts_hbm,
                   f1a, f1b, fbuf, iba, ibb, ftail1, ftail, itail, zrow,
                   cnt2d, acc_sh, sfa, sia, sfb, sib):
    def rows_1d_to_2d(src1, dst2, nrows):
        def blk_body(b, _):
            r0 = b * 8
            e0 = r0 * F
            for r in range(8):
                for t in range(F // 16):
                    dst2[r0 + r, pl.ds(t * 16, 16)] = (
                        src1[pl.ds(e0 + r * F + t * 16, 16)])
            return 0

        lax.fori_loop(0, nrows // 8, blk_body, 0)

    cid = lax.axis_index("c")
    sid = lax.axis_index("s")
    wid = sid * NC + cid
    base = wid * ROWS_MAIN

    lane = lax.iota(jnp.int32, 16)
    ones = jnp.full((16,), 1.0, jnp.float32)

    _zero_2d(cnt2d, 16, C)
    _zero_2d(zrow, 16, F)
    pltpu.sync_copy(zrow, acc_sh.at[pl.ds(sid * 16, 16)])
    plsc.subcore_barrier()

    def start(k, f1, ib, sf, si):
        rowbase = base + k * CHUNK
        pltpu.async_copy(feat_hbm.at[pl.ds(rowbase * F, CHUNK * F)], f1, sf)
        pltpu.async_copy(ids_hbm.at[pl.ds(rowbase, CHUNK)], ib, si)

    def wait(f1, ib, sf, si):
        pltpu.make_async_copy(feat_hbm.at[pl.ds(0, CHUNK * F)], f1, sf).wait()
        pltpu.make_async_copy(ids_hbm.at[pl.ds(0, CHUNK)], ib, si).wait()

    def process(f1, ib):
        rows_1d_to_2d(f1, fbuf, CHUNK)

        def grp_body(g, _):
            iv = ib[pl.ds(g * 16, 16)]
            plsc.addupdate_scatter(cnt2d, [lane, iv], ones)
            return 0

        lax.fori_loop(0, CHUNK // 16, grp_body, 0)
        pltpu.sync_copy(fbuf, acc_sh.at[ib], add=True)

    start(0, f1a, iba, sfa, sia)
    start(1, f1b, ibb, sfb, sib)

    def pair_body(kk, _):
        wait(f1a, iba, sfa, sia)
        process(f1a, iba)
        start(2 * kk + 2, f1a, iba, sfa, sia)
        wait(f1b, ibb, sfb, sib)
        process(f1b, ibb)
        start(2 * kk + 3, f1b, ibb, sfb, sib)
        return 0

    lax.fori_loop(0, NCHUNKS // 2 - 1, pair_body, 0)
    wait(f1a, iba, sfa, sia)
    process(f1a, iba)
    wait(f1b, ibb, sfb, sib)
    process(f1b, ibb)

    @pl.when(wid < NTAIL)
    def _():
        tb = TAIL_BASE + wid * 16
        pltpu.sync_copy(feat_hbm.at[pl.ds(tb * F, 16 * F)], ftail1)
        pltpu.sync_copy(ids_hbm.at[pl.ds(tb, 16)], itail)
        rows_1d_to_2d(ftail1, ftail, 16)
        iv = itail[...]
        plsc.addupdate_scatter(cnt2d, [lane, iv], ones)
        pltpu.sync_copy(ftail, acc_sh.at[itail], add=True)

    plsc.subcore_barrier()

    @pl.when(sid == 0)
    def _():
        pltpu.sync_copy(acc_sh, psums_hbm.at[cid])

    pltpu.sync_copy(cnt2d, pcnts_hbm.at[wid])


def _sc_pass1(feats1d, ids):
    kfn = pl.kernel(
        _sc_pass1_body,
        out_type=(jax.ShapeDtypeStruct((NC, C, F), jnp.float32),
                  jax.ShapeDtypeStruct((NW, 16, C), jnp.float32)),
        mesh=plsc.VectorSubcoreMesh(**_SC_MESH),
        compiler_params=_SC_PARAMS,
        scratch_types=[
            pltpu.VMEM((CHUNK * F,), jnp.float32),
            pltpu.VMEM((CHUNK * F,), jnp.float32),
            pltpu.VMEM((CHUNK, F), jnp.float32),
            pltpu.VMEM((CHUNK,), jnp.int32),
            pltpu.VMEM((CHUNK,), jnp.int32),
            pltpu.VMEM((16 * F,), jnp.float32),
            pltpu.VMEM((16, F), jnp.float32),
            pltpu.VMEM((16,), jnp.int32),
            pltpu.VMEM((16, F), jnp.float32),
            pltpu.VMEM((16, C), jnp.float32),
            pltpu.VMEM_SHARED((C, F), jnp.float32),
            pltpu.SemaphoreType.DMA,
            pltpu.SemaphoreType.DMA,
            pltpu.SemaphoreType.DMA,
            pltpu.SemaphoreType.DMA,
        ],
    )
    return kfn(feats1d, ids)


def _sc_pass2_body(feat_hbm, ids_hbm, means_hbm, ppen_hbm,
                   fba, fbb, iba, ibb, ftail, itail, means_v, pen2d,
                   sfa, sia, sfb, sib):
    cid = lax.axis_index("c")
    sid = lax.axis_index("s")
    wid = sid * NC + cid
    base = wid * ROWS_MAIN

    lane = lax.iota(jnp.int32, 16)

    pltpu.sync_copy(means_hbm, means_v)
    _zero_2d(pen2d, 16, C)

    def process_group(fref, iv, r0):
        rows64 = (lane + r0) * F
        iv64 = iv * F
        accs = [jnp.zeros((16,), jnp.float32) for _ in range(4)]
        for jj in range(F // 4):
            for t in range(4):
                jrot = jnp.bitwise_and(lane + (t * (F // 4) + jj), F - 1)
                fcol = plsc.load_gather(fref, [rows64 + jrot])
                mcol = plsc.load_gather(means_v, [iv64 + jrot])
                d = fcol - mcol
                accs[t] = accs[t] + d * d
        nsq = (accs[0] + accs[1]) + (accs[2] + accs[3]) + 1e-12
        i = plsc.bitcast(nsq, jnp.int32)
        i = jnp.int32(0x5F3759DF) - lax.shift_right_arithmetic(i, jnp.int32(1))
        y = plsc.bitcast(i, jnp.float32)
        for _ in range(3):
            y = y * (1.5 - 0.5 * nsq * y * y)
        dist = nsq * y
        pen = jnp.maximum(dist - DELTA_VAR, 0.0)
        plsc.addupdate_scatter(pen2d, [lane, iv], pen * pen)

    def start(k, fb, ib, sf, si):
        rowbase = base + k * CHUNK
        pltpu.async_copy(feat_hbm.at[pl.ds(rowbase * F, CHUNK * F)], fb, sf)
        pltpu.async_copy(ids_hbm.at[pl.ds(rowbase, CHUNK)], ib, si)

    def wait(fb, ib, sf, si):
        pltpu.make_async_copy(feat_hbm.at[pl.ds(0, CHUNK * F)], fb, sf).wait()
        pltpu.make_async_copy(ids_hbm.at[pl.ds(0, CHUNK)], ib, si).wait()

    def process(fb, ib):
        def grp_body(g, _):
            r0 = g * 16
            process_group(fb, ib[pl.ds(r0, 16)], r0)
            return 0

        lax.fori_loop(0, CHUNK // 16, grp_body, 0)

    start(0, fba, iba, sfa, sia)
    start(1, fbb, ibb, sfb, sib)

    def pair_body(kk, _):
        wait(fba, iba, sfa, sia)
        process(fba, iba)
        start(2 * kk + 2, fba, iba, sfa, sia)
        wait(fbb, ibb, sfb, sib)
        process(fbb, ibb)
        start(2 * kk + 3, fbb, ibb, sfb, sib)
        return 0

    lax.fori_loop(0, NCHUNKS // 2 - 1, pair_body, 0)
    wait(fba, iba, sfa, sia)
    process(fba, iba)
    wait(fbb, ibb, sfb, sib)
    process(fbb, ibb)

    @pl.when(wid < NTAIL)
    def _():
        tb = TAIL_BASE + wid * 16
        pltpu.sync_copy(feat_hbm.at[pl.ds(tb * F, 16 * F)], ftail)
        pltpu.sync_copy(ids_hbm.at[pl.ds(tb, 16)], itail)
        process_group(ftail, itail[...], 0)

    pltpu.sync_copy(pen2d, ppen_hbm.at[wid])


def _sc_pass2(feats1d, ids, means1d):
    kfn = pl.kernel(
        _sc_pass2_body,
        out_type=jax.ShapeDtypeStruct((NW, 16, C), jnp.float32),
        mesh=plsc.VectorSubcoreMesh(**_SC_MESH),
        compiler_params=_SC_PARAMS,
        scratch_types=[
            pltpu.VMEM((CHUNK * F,), jnp.float32),
            pltpu.VMEM((CHUNK * F,), jnp.float32),
            pltpu.VMEM((CHUNK,), jnp.int32),
            pltpu.VMEM((CHUNK,), jnp.int32),
            pltpu.VMEM((16 * F,), jnp.float32),
            pltpu.VMEM((16,), jnp.int32),
            pltpu.VMEM((C * F,), jnp.float32),
            pltpu.VMEM((16, C), jnp.float32),
            pltpu.SemaphoreType.DMA,
            pltpu.SemaphoreType.DMA,
            pltpu.SemaphoreType.DMA,
            pltpu.SemaphoreType.DMA,
        ],
    )
    return kfn(feats1d, ids, means1d)


def _tc_reduce_body(psums_ref, pcnts_ref, means_ref, cnts_ref, dl_ref):
    sums = psums_ref[0] + psums_ref[1]
    cnts = jnp.sum(pcnts_ref[...], axis=(0, 1))
    means = sums / cnts[:, None]
    means_ref[...] = means
    cnts_ref[...] = cnts.reshape(1, C)

    q = jnp.sum(means * means, axis=1)
    g = lax.dot_general(means, means, (((1,), (1,)), ((), ())),
                        preferred_element_type=jnp.float32)
    md2 = jnp.maximum(q[:, None] + q[None, :] - 2.0 * g, 0.0)
    r = lax.broadcasted_iota(jnp.int32, (C, C), 0)
    c = lax.broadcasted_iota(jnp.int32, (C, C), 1)
    eye = (r == c).astype(jnp.float32)
    d = jnp.sqrt(md2 + eye)
    pen = jnp.square(jnp.maximum(DELTA_DIST - d, 0.0)) * (1.0 - eye)
    dl_ref[...] = (jnp.sum(pen) / (C * (C - 1))).reshape(1, 1)


def _tc_reduce(psums, pcnts):
    return pl.pallas_call(
        _tc_reduce_body,
        out_shape=(jax.ShapeDtypeStruct((C, F), jnp.float32),
                   jax.ShapeDtypeStruct((1, C), jnp.float32),
                   jax.ShapeDtypeStruct((1, 1), jnp.float32)),
    )(psums, pcnts)


def _tc_final_body(ppen_ref, cnts_ref, dl_ref, out_ref):
    pen = jnp.sum(ppen_ref[...], axis=(0, 1))
    var_loss = jnp.sum(pen / cnts_ref[0, :]) / C
    out_ref[...] = (var_loss + dl_ref[0, 0]).reshape(1, 1)


def _tc_final(ppen, cnts, dl):
    return pl.pallas_call(
        _tc_final_body,
        out_shape=jax.ShapeDtypeStruct((1, 1), jnp.float32),
    )(ppen, cnts, dl)


def kernel(features, labels):
    feats1d = jnp.reshape(features, (-1,))
    ids = labels[:, 1]
    psums, pcnts = _sc_pass1(feats1d, ids)
    means, cnts, dl = _tc_reduce(psums, pcnts)
    ppen = _sc_pass2(feats1d, ids, jnp.reshape(means, (-1,)))
    out = _tc_final(ppen, cnts, dl)
    return out[0, 0]


# speedup vs baseline: 1.6816x; 1.0324x over previous
import jax
import jax.numpy as jnp
from jax import lax
from jax.experimental import pallas as pl
from jax.experimental.pallas import tpu as pltpu
from jax.experimental.pallas import tpu_sc as plsc

N = 200000
F = 64
C = 256
DELTA_VAR = 0.5
DELTA_DIST = 3.0

NC = 2
NS = 16
NW = NC * NS

ROWS_MAIN = 6240
CHUNK = 208
NCHUNKS = ROWS_MAIN // CHUNK
TAIL_BASE = NW * ROWS_MAIN
NTAIL = (N - TAIL_BASE) // 16

_SC_PARAMS = pltpu.CompilerParams(use_tc_tiling_on_sc=False,
                                  needs_layout_passes=False)
_SC_MESH = dict(core_axis_name="c", subcore_axis_name="s")


def _zero_1d(ref, n):
    zeros = jnp.zeros((16,), jnp.float32)

    def body(i, _):
        ref[pl.ds(i * 16, 16)] = zeros
        return 0

    lax.fori_loop(0, n // 16, body, 0)


def _zero_2d(ref, rows, cols):
    zeros = jnp.zeros((16,), jnp.float32)

    def body(r, _):
        for j in range(cols // 16):
            ref[r, pl.ds(j * 16, 16)] = zeros
        return 0

    lax.fori_loop(0, rows, body, 0)


def _sc_pass1_body(feat_hbm, ids_hbm, psums_hbm, pcnts_hbm,
                   fa, fb, iba, ibb, ftail, itail, zrow,
                   cnt2d, acc_sh, sfa, sia, sfb, sib):
    cid = lax.axis_index("c")
    sid = lax.axis_index("s")
    wid = sid * NC + cid
    base = wid * ROWS_MAIN

    lane = lax.iota(jnp.int32, 16)
    ones = jnp.full((16,), 1.0, jnp.float32)

    _zero_2d(cnt2d, 16, C)
    _zero_2d(zrow, 16, F)
    pltpu.sync_copy(zrow, acc_sh.at[pl.ds(sid * 16, 16)])
    plsc.subcore_barrier()

    def start(k, f, ib, sf, si):
        rowbase = base + k * CHUNK
        pltpu.async_copy(feat_hbm.at[pl.ds(rowbase, CHUNK)], f, sf)
        pltpu.async_copy(ids_hbm.at[pl.ds(rowbase, CHUNK)], ib, si)

    def wait(f, ib, sf, si):
        pltpu.make_async_copy(feat_hbm.at[pl.ds(0, CHUNK)], f, sf).wait()
        pltpu.make_async_copy(ids_hbm.at[pl.ds(0, CHUNK)], ib, si).wait()

    def process(f, ib):
        def grp_body(g, _):
            iv = ib[pl.ds(g * 16, 16)]
            plsc.addupdate_scatter(cnt2d, [lane, iv], ones)
            return 0

        lax.fori_loop(0, CHUNK // 16, grp_body, 0)
        pltpu.sync_copy(f, acc_sh.at[ib], add=True)

    start(0, fa, iba, sfa, sia)
    start(1, fb, ibb, sfb, sib)

    def pair_body(kk, _):
        wait(fa, iba, sfa, sia)
        process(fa, iba)
        start(2 * kk + 2, fa, iba, sfa, sia)
        wait(fb, ibb, sfb, sib)
        process(fb, ibb)
        start(2 * kk + 3, fb, ibb, sfb, sib)
        return 0

    lax.fori_loop(0, NCHUNKS // 2 - 1, pair_body, 0)
    wait(fa, iba, sfa, sia)
    process(fa, iba)
    wait(fb, ibb, sfb, sib)
    process(fb, ibb)

    @pl.when(wid < NTAIL)
    def _():
        tb = TAIL_BASE + wid * 16
        pltpu.sync_copy(feat_hbm.at[pl.ds(tb, 16)], ftail)
        pltpu.sync_copy(ids_hbm.at[pl.ds(tb, 16)], itail)
        iv = itail[...]
        plsc.addupdate_scatter(cnt2d, [lane, iv], ones)
        pltpu.sync_copy(ftail, acc_sh.at[itail], add=True)

    plsc.subcore_barrier()

    @pl.when(sid == 0)
    def _():
        pltpu.sync_copy(acc_sh, psums_hbm.at[cid])

    pltpu.sync_copy(cnt2d, pcnts_hbm.at[wid])


def _sc_pass1(features, ids):
    kfn = pl.kernel(
        _sc_pass1_body,
        out_type=(jax.ShapeDtypeStruct((NC, C, F), jnp.float32),
                  jax.ShapeDtypeStruct((NW, 16, C), jnp.float32)),
        mesh=plsc.VectorSubcoreMesh(**_SC_MESH),
        compiler_params=_SC_PARAMS,
        scratch_types=[
            pltpu.VMEM((CHUNK, F), jnp.float32),
            pltpu.VMEM((CHUNK, F), jnp.float32),
            pltpu.VMEM((CHUNK,), jnp.int32),
            pltpu.VMEM((CHUNK,), jnp.int32),
            pltpu.VMEM((16, F), jnp.float32),
            pltpu.VMEM((16,), jnp.int32),
            pltpu.VMEM((16, F), jnp.float32),
            pltpu.VMEM((16, C), jnp.float32),
            pltpu.VMEM_SHARED((C, F), jnp.float32),
            pltpu.SemaphoreType.DMA,
            pltpu.SemaphoreType.DMA,
            pltpu.SemaphoreType.DMA,
            pltpu.SemaphoreType.DMA,
        ],
    )
    return kfn(features, ids)


def _sc_pass2_body(feat_hbm, ids_hbm, means_hbm, ppen_hbm,
                   fba, fbb, iba, ibb, ftail, itail, means_v, pen2d,
                   sfa, sia, sfb, sib):
    cid = lax.axis_index("c")
    sid = lax.axis_index("s")
    wid = sid * NC + cid
    base = wid * ROWS_MAIN

    lane = lax.iota(jnp.int32, 16)

    pltpu.sync_copy(means_hbm, means_v)
    _zero_2d(pen2d, 16, C)

    def process_group(fref, iv, r0):
        rows = lane + r0
        iv64 = iv * F
        accs = [jnp.zeros((16,), jnp.float32) for _ in range(4)]
        for jj in range(F // 4):
            for t in range(4):
                jrot = jnp.bitwise_and(lane + (t * (F // 4) + jj), F - 1)
                fcol = plsc.load_gather(fref, [rows, jrot])
                mcol = plsc.load_gather(means_v, [iv64 + jrot])
                d = fcol - mcol
                accs[t] = accs[t] + d * d
        nsq = (accs[0] + accs[1]) + (accs[2] + accs[3]) + 1e-12
        i = plsc.bitcast(nsq, jnp.int32)
        i = jnp.int32(0x5F3759DF) - lax.shift_right_arithmetic(i, jnp.int32(1))
        y = plsc.bitcast(i, jnp.float32)
        for _ in range(3):
            y = y * (1.5 - 0.5 * nsq * y * y)
        dist = nsq * y
        pen = jnp.maximum(dist - DELTA_VAR, 0.0)
        plsc.addupdate_scatter(pen2d, [lane, iv], pen * pen)

    def start(k, fb, ib, sf, si):
        rowbase = base + k * CHUNK
        pltpu.async_copy(feat_hbm.at[pl.ds(rowbase, CHUNK)], fb, sf)
        pltpu.async_copy(ids_hbm.at[pl.ds(rowbase, CHUNK)], ib, si)

    def wait(fb, ib, sf, si):
        pltpu.make_async_copy(feat_hbm.at[pl.ds(0, CHUNK)], fb, sf).wait()
        pltpu.make_async_copy(ids_hbm.at[pl.ds(0, CHUNK)], ib, si).wait()

    def process(fb, ib):
        def grp_body(g, _):
            r0 = g * 16
            process_group(fb, ib[pl.ds(r0, 16)], r0)
            return 0

        lax.fori_loop(0, CHUNK // 16, grp_body, 0)

    start(0, fba, iba, sfa, sia)
    start(1, fbb, ibb, sfb, sib)

    def pair_body(kk, _):
        wait(fba, iba, sfa, sia)
        process(fba, iba)
        start(2 * kk + 2, fba, iba, sfa, sia)
        wait(fbb, ibb, sfb, sib)
        process(fbb, ibb)
        start(2 * kk + 3, fbb, ibb, sfb, sib)
        return 0

    lax.fori_loop(0, NCHUNKS // 2 - 1, pair_body, 0)
    wait(fba, iba, sfa, sia)
    process(fba, iba)
    wait(fbb, ibb, sfb, sib)
    process(fbb, ibb)

    @pl.when(wid < NTAIL)
    def _():
        tb = TAIL_BASE + wid * 16
        pltpu.sync_copy(feat_hbm.at[pl.ds(tb, 16)], ftail)
        pltpu.sync_copy(ids_hbm.at[pl.ds(tb, 16)], itail)
        process_group(ftail, itail[...], 0)

    pltpu.sync_copy(pen2d, ppen_hbm.at[wid])


def _sc_pass2(features, ids, means1d):
    kfn = pl.kernel(
        _sc_pass2_body,
        out_type=jax.ShapeDtypeStruct((NW, 16, C), jnp.float32),
        mesh=plsc.VectorSubcoreMesh(**_SC_MESH),
        compiler_params=_SC_PARAMS,
        scratch_types=[
            pltpu.VMEM((CHUNK, F), jnp.float32),
            pltpu.VMEM((CHUNK, F), jnp.float32),
            pltpu.VMEM((CHUNK,), jnp.int32),
            pltpu.VMEM((CHUNK,), jnp.int32),
            pltpu.VMEM((16, F), jnp.float32),
            pltpu.VMEM((16,), jnp.int32),
            pltpu.VMEM((C * F,), jnp.float32),
            pltpu.VMEM((16, C), jnp.float32),
            pltpu.SemaphoreType.DMA,
            pltpu.SemaphoreType.DMA,
            pltpu.SemaphoreType.DMA,
            pltpu.SemaphoreType.DMA,
        ],
    )
    return kfn(features, ids, means1d)


def _tc_reduce_body(psums_ref, pcnts_ref, means_ref, cnts_ref, dl_ref):
    sums = psums_ref[0] + psums_ref[1]
    cnts = jnp.sum(pcnts_ref[...], axis=(0, 1))
    means = sums / cnts[:, None]
    means_ref[...] = means
    cnts_ref[...] = cnts.reshape(1, C)

    q = jnp.sum(means * means, axis=1)
    g = lax.dot_general(means, means, (((1,), (1,)), ((), ())),
                        preferred_element_type=jnp.float32)
    md2 = jnp.maximum(q[:, None] + q[None, :] - 2.0 * g, 0.0)
    r = lax.broadcasted_iota(jnp.int32, (C, C), 0)
    c = lax.broadcasted_iota(jnp.int32, (C, C), 1)
    eye = (r == c).astype(jnp.float32)
    d = jnp.sqrt(md2 + eye)
    pen = jnp.square(jnp.maximum(DELTA_DIST - d, 0.0)) * (1.0 - eye)
    dl_ref[...] = (jnp.sum(pen) / (C * (C - 1))).reshape(1, 1)


def _tc_reduce(psums, pcnts):
    return pl.pallas_call(
        _tc_reduce_body,
        out_shape=(jax.ShapeDtypeStruct((C, F), jnp.float32),
                   jax.ShapeDtypeStruct((1, C), jnp.float32),
                   jax.ShapeDtypeStruct((1, 1), jnp.float32)),
    )(psums, pcnts)


def _tc_final_body(ppen_ref, cnts_ref, dl_ref, out_ref):
    pen = jnp.sum(ppen_ref[...], axis=(0, 1))
    var_loss = jnp.sum(pen / cnts_ref[0, :]) / C
    out_ref[...] = (var_loss + dl_ref[0, 0]).reshape(1, 1)


def _tc_final(ppen, cnts, dl):
    return pl.pallas_call(
        _tc_final_body,
        out_shape=jax.ShapeDtypeStruct((1, 1), jnp.float32),
    )(ppen, cnts, dl)


def kernel(features, labels):
    ids = labels[:, 1]
    psums, pcnts = _sc_pass1(features, ids)
    means, cnts, dl = _tc_reduce(psums, pcnts)
    ppen = _sc_pass2(features, ids, jnp.reshape(means, (-1,)))
    out = _tc_final(ppen, cnts, dl)
    return out[0, 0]
